# baseline (device time: 196899 ns/iter reference)
import functools

import jax
import jax.numpy as jnp
from jax import lax
from jax.experimental import pallas as pl
from jax.experimental.pallas import tpu as pltpu

_DeviceIdType = getattr(pl, "DeviceIdType", None) or pltpu.DeviceIdType
_sem_signal = getattr(pl, "semaphore_signal", None) or pltpu.semaphore_signal
_sem_wait = getattr(pl, "semaphore_wait", None) or pltpu.semaphore_wait
_run_scoped = getattr(pl, "run_scoped", None) or pltpu.run_scoped
_CompilerParams = getattr(pltpu, "CompilerParams", None) or getattr(
    pltpu, "TPUCompilerParams"
)

N_DEV = 16
H_PER = 8
SQ = 2048
SKV = 2048
DH = 128
DMODEL = 1024
SCALE = 0.08838834764831843
QT = 512
CHUNK = SQ // N_DEV
HALF = DMODEL // 2
NSTEP = N_DEV - 1
VLIM = 100 * 1024 * 1024


def _ring_pos(p):
    z = p // 4
    k = lax.rem(p, 4)
    return jnp.where(
        k == 0, z,
        jnp.where(k == 3, 7 - z, jnp.where(k == 2, 8 + z, 15 - z)),
    )


def _ring_to_log(rr):
    seg = rr // 4
    z = jnp.where(
        seg == 0, rr,
        jnp.where(seg == 1, 7 - rr, jnp.where(seg == 2, rr - 8, 15 - rr)),
    )
    k = jnp.where(
        seg == 0, 0, jnp.where(seg == 1, 3, jnp.where(seg == 2, 2, 1))
    )
    return 4 * z + k


def _attn_body(x_ref, wq_ref, k_hbm, v_hbm, ctx_ref, kbuf, vbuf, sk, sv):
    h = pl.program_id(0)
    p = lax.axis_index("i")
    ghead = p * H_PER + h
    slot = lax.rem(h, 2)
    nslot = lax.rem(h + 1, 2)

    def copies(head, sl):
        ck = pltpu.make_async_copy(k_hbm.at[0, :, head, :], kbuf.at[sl], sk.at[sl])
        cv = pltpu.make_async_copy(v_hbm.at[0, :, head, :], vbuf.at[sl], sv.at[sl])
        return ck, cv

    @pl.when(h == 0)
    def _():
        a, b = copies(ghead, slot)
        a.start()
        b.start()

    @pl.when(h < H_PER - 1)
    def _():
        a, b = copies(ghead + 1, nslot)
        a.start()
        b.start()

    q = jnp.dot(x_ref[...], wq_ref[...], preferred_element_type=jnp.float32)

    a, b = copies(ghead, slot)
    a.wait()
    b.wait()
    k = kbuf[slot].astype(jnp.bfloat16)
    v = vbuf[slot].astype(jnp.bfloat16)
    k0 = k[0:64]
    v0 = v[0:64]
    q = q.astype(jnp.bfloat16)

    def dotT(a_, b_):
        return lax.dot_general(a_, b_, (((1,), (1,)), ((), ())),
                               preferred_element_type=jnp.float32)

    for rho in range(3):
        qbs = [bq for bq in range(SQ // 64) if bq % 3 == rho]
        kbs = [bk for bk in range(SKV // 64) if bk % 3 == (3 - rho) % 3]
        qr = jnp.concatenate([q[bq * 64:(bq + 1) * 64] for bq in qbs], axis=0)
        ks = jnp.concatenate([k[bk * 64:(bk + 1) * 64] for bk in kbs], axis=0)
        vs = jnp.concatenate([v[bk * 64:(bk + 1) * 64] for bk in kbs], axis=0)
        parts = [dotT(qr, ks) * SCALE]
        if rho != 0:
            parts.append(dotT(qr, k0) * SCALE)
            parts.append(jnp.concatenate(
                [dotT(q[bq * 64:(bq + 1) * 64], k[bq * 64:(bq + 1) * 64])
                 for bq in qbs], axis=0) * SCALE)
        s = jnp.concatenate(parts, axis=1)
        m = jnp.max(s, axis=1, keepdims=True)
        w = jnp.exp(s - m)
        w = (w / jnp.sum(w, axis=1, keepdims=True)).astype(jnp.bfloat16)
        ncs = len(kbs) * 64
        ctx_main = jnp.dot(w[:, 0:ncs], vs, preferred_element_type=jnp.float32)
        if rho != 0:
            ctx_main = ctx_main + jnp.dot(w[:, ncs:ncs + 64], v0,
                                          preferred_element_type=jnp.float32)
            wd = w[:, ncs + 64:]
            for i, bq in enumerate(qbs):
                ctx_ref[0, bq * 64:(bq + 1) * 64, :] = (
                    ctx_main[i * 64:(i + 1) * 64]
                    + jnp.dot(wd[i * 64:(i + 1) * 64],
                              v[bq * 64:(bq + 1) * 64],
                              preferred_element_type=jnp.float32)
                ).astype(jnp.bfloat16)
        else:
            for i, bq in enumerate(qbs):
                ctx_ref[0, bq * 64:(bq + 1) * 64, :] = \
                    ctx_main[i * 64:(i + 1) * 64].astype(jnp.bfloat16)


def _ar_body(ctx_ref, wo_ref, out_ref, part,
             sbR, sbL, rsR, rsL, agR, agL, abR, abL,
             ssR, ssL, rsmR, rsmL, agmR, agmL):
    p = lax.axis_index("i")
    r = _ring_pos(p)
    nxt = _ring_to_log(lax.rem(r + 1, N_DEV))
    prv = _ring_to_log(lax.rem(r + N_DEV - 1, N_DEV))

    barrier = pltpu.get_barrier_semaphore()
    for nbr in (prv, nxt):
        _sem_signal(barrier, inc=1, device_id=(nbr,),
                    device_id_type=_DeviceIdType.MESH)
    _sem_wait(barrier, 2)

    acc = jnp.dot(ctx_ref[0], wo_ref[0], preferred_element_type=jnp.float32)
    for h in range(1, H_PER):
        acc = acc + jnp.dot(ctx_ref[h], wo_ref[h],
                            preferred_element_type=jnp.float32)
    part[...] = acc

    def rows(c):
        return pl.ds(c * CHUNK, CHUNK)

    def cR(s):
        return lax.rem(r - s + N_DEV, N_DEV)

    def cL(s):
        return lax.rem(r + s, N_DEV)

    def rs_rdma(s, right):
        if right:
            return pltpu.make_async_remote_copy(
                src_ref=sbR.at[s % 2], dst_ref=rsR.at[s],
                send_sem=ssR.at[s % 2], recv_sem=rsmR.at[s],
                device_id=(nxt,), device_id_type=_DeviceIdType.MESH)
        return pltpu.make_async_remote_copy(
            src_ref=sbL.at[s % 2], dst_ref=rsL.at[s],
            send_sem=ssL.at[s % 2], recv_sem=rsmL.at[s],
            device_id=(prv,), device_id_type=_DeviceIdType.MESH)

    sbR[0] = part[rows(cR(0)), 0:HALF].astype(jnp.bfloat16)
    sbL[0] = part[rows(cL(0)), HALF:DMODEL].astype(jnp.bfloat16)
    rs_rdma(0, True).start()
    rs_rdma(0, False).start()
    for s in range(1, NSTEP):
        rs_rdma(s - 1, True).wait_recv()
        rs_rdma(s - 1, False).wait_recv()
        if s >= 2:
            rs_rdma(s - 2, True).wait_send()
            rs_rdma(s - 2, False).wait_send()
        sbR[s % 2] = (rsR[s - 1].astype(jnp.float32)
                      + part[rows(cR(s)), 0:HALF]).astype(jnp.bfloat16)
        sbL[s % 2] = (rsL[s - 1].astype(jnp.float32)
                      + part[rows(cL(s)), HALF:DMODEL]).astype(jnp.bfloat16)
        rs_rdma(s, True).start()
        rs_rdma(s, False).start()
    rs_rdma(NSTEP - 1, True).wait_recv()
    rs_rdma(NSTEP - 1, False).wait_recv()
    for s in (NSTEP - 2, NSTEP - 1):
        rs_rdma(s, True).wait_send()
        rs_rdma(s, False).wait_send()

    redR = rsR[NSTEP - 1].astype(jnp.float32) + part[rows(cR(NSTEP)), 0:HALF]
    redL = (rsL[NSTEP - 1].astype(jnp.float32)
            + part[rows(cL(NSTEP)), HALF:DMODEL])
    abR[...] = redR.astype(jnp.bfloat16)
    abL[...] = redL.astype(jnp.bfloat16)
    out_ref[rows(cR(NSTEP)), 0:HALF] = redR
    out_ref[rows(cL(NSTEP)), HALF:DMODEL] = redL

    def ag_rdma(s, right):
        if right:
            return pltpu.make_async_remote_copy(
                src_ref=(abR if s == 0 else agR.at[s - 1]), dst_ref=agR.at[s],
                send_sem=ssR.at[s % 2], recv_sem=agmR.at[s],
                device_id=(nxt,), device_id_type=_DeviceIdType.MESH)
        return pltpu.make_async_remote_copy(
            src_ref=(abL if s == 0 else agL.at[s - 1]), dst_ref=agL.at[s],
            send_sem=ssL.at[s % 2], recv_sem=agmL.at[s],
            device_id=(prv,), device_id_type=_DeviceIdType.MESH)

    ag_rdma(0, True).start()
    ag_rdma(0, False).start()
    for s in range(NSTEP):
        ag_rdma(s, True).wait_recv()
        ag_rdma(s, False).wait_recv()
        if s < NSTEP - 1:
            if s >= 1:
                ag_rdma(s - 1, True).wait_send()
                ag_rdma(s - 1, False).wait_send()
            ag_rdma(s + 1, True).start()
            ag_rdma(s + 1, False).start()
        out_ref[rows(cR(s)), 0:HALF] = agR[s].astype(jnp.float32)
        out_ref[rows(cL(s)), HALF:DMODEL] = agL[s].astype(jnp.float32)
    for s in (NSTEP - 2, NSTEP - 1):
        ag_rdma(s, True).wait_send()
        ag_rdma(s, False).wait_send()

    @functools.partial(_run_scoped, sem2=pltpu.SemaphoreType.REGULAR)
    def _(sem2):
        for nbr in (prv, nxt):
            _sem_signal(sem2, inc=1, device_id=(nbr,),
                        device_id_type=_DeviceIdType.MESH)
        _sem_wait(sem2, 2)


def kernel(x, Wq, K_ext, V_ext, Wo):
    x2 = x.reshape(SQ, DMODEL)

    ctx = pl.pallas_call(
        _attn_body,
        grid=(H_PER,),
        out_shape=jax.ShapeDtypeStruct((H_PER, SQ, DH), jnp.bfloat16),
        in_specs=[
            pl.BlockSpec((SQ, DMODEL), lambda h: (0, 0)),
            pl.BlockSpec((DMODEL, DH), lambda h: (0, h)),
            pl.BlockSpec(memory_space=pl.ANY),
            pl.BlockSpec(memory_space=pl.ANY),
        ],
        out_specs=pl.BlockSpec((1, SQ, DH), lambda h: (h, 0, 0)),
        scratch_shapes=[
            pltpu.VMEM((2, SKV, DH), jnp.float32),
            pltpu.VMEM((2, SKV, DH), jnp.float32),
            pltpu.SemaphoreType.DMA((2,)),
            pltpu.SemaphoreType.DMA((2,)),
        ],
        compiler_params=_CompilerParams(vmem_limit_bytes=VLIM),
    )(x2.astype(jnp.bfloat16), Wq.astype(jnp.bfloat16), K_ext, V_ext)

    out2 = pl.pallas_call(
        _ar_body,
        out_shape=jax.ShapeDtypeStruct((SQ, DMODEL), jnp.float32),
        in_specs=[
            pl.BlockSpec(memory_space=pltpu.MemorySpace.VMEM),
            pl.BlockSpec(memory_space=pltpu.MemorySpace.VMEM),
        ],
        out_specs=pl.BlockSpec(memory_space=pltpu.MemorySpace.VMEM),
        scratch_shapes=[
            pltpu.VMEM((SQ, DMODEL), jnp.float32),
            pltpu.VMEM((2, CHUNK, HALF), jnp.bfloat16),
            pltpu.VMEM((2, CHUNK, HALF), jnp.bfloat16),
            pltpu.VMEM((NSTEP, CHUNK, HALF), jnp.bfloat16),
            pltpu.VMEM((NSTEP, CHUNK, HALF), jnp.bfloat16),
            pltpu.VMEM((NSTEP, CHUNK, HALF), jnp.bfloat16),
            pltpu.VMEM((NSTEP, CHUNK, HALF), jnp.bfloat16),
            pltpu.VMEM((CHUNK, HALF), jnp.bfloat16),
            pltpu.VMEM((CHUNK, HALF), jnp.bfloat16),
            pltpu.SemaphoreType.DMA((2,)),
            pltpu.SemaphoreType.DMA((2,)),
            pltpu.SemaphoreType.DMA((NSTEP,)),
            pltpu.SemaphoreType.DMA((NSTEP,)),
            pltpu.SemaphoreType.DMA((NSTEP,)),
            pltpu.SemaphoreType.DMA((NSTEP,)),
        ],
        compiler_params=_CompilerParams(
            collective_id=0, vmem_limit_bytes=VLIM
        ),
    )(ctx, Wo.reshape(H_PER, DH, DMODEL).astype(jnp.bfloat16))

    return out2.reshape(1, SQ, DMODEL)


# device time: 182491 ns/iter; 1.0790x vs baseline; 1.0790x over previous
import functools

import jax
import jax.numpy as jnp
from jax import lax
from jax.experimental import pallas as pl
from jax.experimental.pallas import tpu as pltpu

_DeviceIdType = getattr(pl, "DeviceIdType", None) or pltpu.DeviceIdType
_sem_signal = getattr(pl, "semaphore_signal", None) or pltpu.semaphore_signal
_sem_wait = getattr(pl, "semaphore_wait", None) or pltpu.semaphore_wait
_run_scoped = getattr(pl, "run_scoped", None) or pltpu.run_scoped
_CompilerParams = getattr(pltpu, "CompilerParams", None) or getattr(
    pltpu, "TPUCompilerParams"
)

N_DEV = 16
H_PER = 8
SQ = 2048
SKV = 2048
DH = 128
DMODEL = 1024
SCALE = 0.08838834764831843
QT = 512
CHUNK = SQ // N_DEV
HALF = DMODEL // 2
NSTEP = N_DEV - 1
VLIM = 100 * 1024 * 1024


def _ring_pos(p):
    z = p // 4
    k = lax.rem(p, 4)
    return jnp.where(
        k == 0, z,
        jnp.where(k == 3, 7 - z, jnp.where(k == 2, 8 + z, 15 - z)),
    )


def _ring_to_log(rr):
    seg = rr // 4
    z = jnp.where(
        seg == 0, rr,
        jnp.where(seg == 1, 7 - rr, jnp.where(seg == 2, rr - 8, 15 - rr)),
    )
    k = jnp.where(
        seg == 0, 0, jnp.where(seg == 1, 3, jnp.where(seg == 2, 2, 1))
    )
    return 4 * z + k


def _attn_body(x_ref, wq_ref, k_hbm, v_hbm, ctx_ref, kbuf, vbuf, sk, sv):
    h = pl.program_id(0)
    p = lax.axis_index("i")
    ghead = p * H_PER + h
    slot = lax.rem(h, 2)
    nslot = lax.rem(h + 1, 2)

    def copies(head, sl):
        ck = pltpu.make_async_copy(k_hbm.at[0, :, head, :], kbuf.at[sl], sk.at[sl])
        cv = pltpu.make_async_copy(v_hbm.at[0, :, head, :], vbuf.at[sl], sv.at[sl])
        return ck, cv

    @pl.when(h == 0)
    def _():
        a, b = copies(ghead, slot)
        a.start()
        b.start()

    @pl.when(h < H_PER - 1)
    def _():
        a, b = copies(ghead + 1, nslot)
        a.start()
        b.start()

    q = jnp.dot(x_ref[...], wq_ref[...], preferred_element_type=jnp.float32)

    a, b = copies(ghead, slot)
    a.wait()
    b.wait()
    k = kbuf[slot]
    v = vbuf[slot]
    k0 = k[0:64]
    v0 = v[0:64]

    def dotT(a_, b_):
        return lax.dot_general(a_, b_, (((1,), (1,)), ((), ())),
                               preferred_element_type=jnp.float32)

    for rho in range(3):
        qbs = [bq for bq in range(SQ // 64) if bq % 3 == rho]
        kbs = [bk for bk in range(SKV // 64) if bk % 3 == (3 - rho) % 3]
        qr = jnp.concatenate([q[bq * 64:(bq + 1) * 64] for bq in qbs], axis=0)
        ks = jnp.concatenate([k[bk * 64:(bk + 1) * 64] for bk in kbs], axis=0)
        vs = jnp.concatenate([v[bk * 64:(bk + 1) * 64] for bk in kbs], axis=0)
        parts = [dotT(qr, ks) * SCALE]
        if rho != 0:
            parts.append(dotT(qr, k0) * SCALE)
            parts.append(jnp.concatenate(
                [dotT(q[bq * 64:(bq + 1) * 64], k[bq * 64:(bq + 1) * 64])
                 for bq in qbs], axis=0) * SCALE)
        s = jnp.concatenate(parts, axis=1)
        m = jnp.max(s, axis=1, keepdims=True)
        w = jnp.exp(s - m)
        w = w / jnp.sum(w, axis=1, keepdims=True)
        ncs = len(kbs) * 64
        ctx_main = jnp.dot(w[:, 0:ncs], vs, preferred_element_type=jnp.float32)
        if rho != 0:
            ctx_main = ctx_main + jnp.dot(w[:, ncs:ncs + 64], v0,
                                          preferred_element_type=jnp.float32)
            wd = w[:, ncs + 64:]
            for i, bq in enumerate(qbs):
                ctx_ref[0, bq * 64:(bq + 1) * 64, :] = (
                    ctx_main[i * 64:(i + 1) * 64]
                    + jnp.dot(wd[i * 64:(i + 1) * 64],
                              v[bq * 64:(bq + 1) * 64],
                              preferred_element_type=jnp.float32))
        else:
            for i, bq in enumerate(qbs):
                ctx_ref[0, bq * 64:(bq + 1) * 64, :] = \
                    ctx_main[i * 64:(i + 1) * 64]


def _ar_body(ctx_ref, wo_ref, out_ref, part,
             sbR, sbL, rsR, rsL, agR, agL, abR, abL,
             ssR, ssL, rsmR, rsmL, agmR, agmL):
    p = lax.axis_index("i")
    r = _ring_pos(p)
    nxt = _ring_to_log(lax.rem(r + 1, N_DEV))
    prv = _ring_to_log(lax.rem(r + N_DEV - 1, N_DEV))

    barrier = pltpu.get_barrier_semaphore()
    for nbr in (prv, nxt):
        _sem_signal(barrier, inc=1, device_id=(nbr,),
                    device_id_type=_DeviceIdType.MESH)
    _sem_wait(barrier, 2)

    def rows(c):
        return pl.ds(c * CHUNK, CHUNK)

    def proj_chunk(c):
        rws = rows(c)
        accc = jnp.dot(ctx_ref[0, rws, :], wo_ref[0],
                       preferred_element_type=jnp.float32)
        for h in range(1, H_PER):
            accc = accc + jnp.dot(ctx_ref[h, rws, :], wo_ref[h],
                                  preferred_element_type=jnp.float32)
        part[rws, :] = accc

    proj_chunk(r)

    def cR(s):
        return lax.rem(r - s + N_DEV, N_DEV)

    def cL(s):
        return lax.rem(r + s, N_DEV)

    def rs_rdma(s, right):
        if right:
            return pltpu.make_async_remote_copy(
                src_ref=sbR.at[s % 2], dst_ref=rsR.at[s],
                send_sem=ssR.at[s % 2], recv_sem=rsmR.at[s],
                device_id=(nxt,), device_id_type=_DeviceIdType.MESH)
        return pltpu.make_async_remote_copy(
            src_ref=sbL.at[s % 2], dst_ref=rsL.at[s],
            send_sem=ssL.at[s % 2], recv_sem=rsmL.at[s],
            device_id=(prv,), device_id_type=_DeviceIdType.MESH)

    sbR[0] = part[rows(cR(0)), 0:HALF].astype(jnp.bfloat16)
    sbL[0] = part[rows(cL(0)), HALF:DMODEL].astype(jnp.bfloat16)
    rs_rdma(0, True).start()
    rs_rdma(0, False).start()
    for s in range(1, NSTEP):
        if s <= 7:
            proj_chunk(lax.rem(r - s + N_DEV, N_DEV))
            proj_chunk(lax.rem(r + s, N_DEV))
        elif s == 8:
            proj_chunk(lax.rem(r + 8, N_DEV))
        rs_rdma(s - 1, True).wait_recv()
        rs_rdma(s - 1, False).wait_recv()
        if s >= 2:
            rs_rdma(s - 2, True).wait_send()
            rs_rdma(s - 2, False).wait_send()
        sbR[s % 2] = (rsR[s - 1].astype(jnp.float32)
                      + part[rows(cR(s)), 0:HALF]).astype(jnp.bfloat16)
        sbL[s % 2] = (rsL[s - 1].astype(jnp.float32)
                      + part[rows(cL(s)), HALF:DMODEL]).astype(jnp.bfloat16)
        rs_rdma(s, True).start()
        rs_rdma(s, False).start()
    rs_rdma(NSTEP - 1, True).wait_recv()
    rs_rdma(NSTEP - 1, False).wait_recv()
    for s in (NSTEP - 2, NSTEP - 1):
        rs_rdma(s, True).wait_send()
        rs_rdma(s, False).wait_send()

    redR = rsR[NSTEP - 1].astype(jnp.float32) + part[rows(cR(NSTEP)), 0:HALF]
    redL = (rsL[NSTEP - 1].astype(jnp.float32)
            + part[rows(cL(NSTEP)), HALF:DMODEL])
    abR[...] = redR.astype(jnp.bfloat16)
    abL[...] = redL.astype(jnp.bfloat16)
    out_ref[rows(cR(NSTEP)), 0:HALF] = redR
    out_ref[rows(cL(NSTEP)), HALF:DMODEL] = redL

    def ag_rdma(s, right):
        if right:
            return pltpu.make_async_remote_copy(
                src_ref=(abR if s == 0 else agR.at[s - 1]), dst_ref=agR.at[s],
                send_sem=ssR.at[s % 2], recv_sem=agmR.at[s],
                device_id=(nxt,), device_id_type=_DeviceIdType.MESH)
        return pltpu.make_async_remote_copy(
            src_ref=(abL if s == 0 else agL.at[s - 1]), dst_ref=agL.at[s],
            send_sem=ssL.at[s % 2], recv_sem=agmL.at[s],
            device_id=(prv,), device_id_type=_DeviceIdType.MESH)

    ag_rdma(0, True).start()
    ag_rdma(0, False).start()
    for s in range(NSTEP):
        ag_rdma(s, True).wait_recv()
        ag_rdma(s, False).wait_recv()
        if s < NSTEP - 1:
            if s >= 1:
                ag_rdma(s - 1, True).wait_send()
                ag_rdma(s - 1, False).wait_send()
            ag_rdma(s + 1, True).start()
            ag_rdma(s + 1, False).start()
        out_ref[rows(cR(s)), 0:HALF] = agR[s].astype(jnp.float32)
        out_ref[rows(cL(s)), HALF:DMODEL] = agL[s].astype(jnp.float32)
    for s in (NSTEP - 2, NSTEP - 1):
        ag_rdma(s, True).wait_send()
        ag_rdma(s, False).wait_send()

    @functools.partial(_run_scoped, sem2=pltpu.SemaphoreType.REGULAR)
    def _(sem2):
        for nbr in (prv, nxt):
            _sem_signal(sem2, inc=1, device_id=(nbr,),
                        device_id_type=_DeviceIdType.MESH)
        _sem_wait(sem2, 2)


def kernel(x, Wq, K_ext, V_ext, Wo):
    x2 = x.reshape(SQ, DMODEL)

    ctx = pl.pallas_call(
        _attn_body,
        grid=(H_PER,),
        out_shape=jax.ShapeDtypeStruct((H_PER, SQ, DH), jnp.float32),
        in_specs=[
            pl.BlockSpec((SQ, DMODEL), lambda h: (0, 0)),
            pl.BlockSpec((DMODEL, DH), lambda h: (0, h)),
            pl.BlockSpec(memory_space=pl.ANY),
            pl.BlockSpec(memory_space=pl.ANY),
        ],
        out_specs=pl.BlockSpec((1, SQ, DH), lambda h: (h, 0, 0)),
        scratch_shapes=[
            pltpu.VMEM((2, SKV, DH), jnp.float32),
            pltpu.VMEM((2, SKV, DH), jnp.float32),
            pltpu.SemaphoreType.DMA((2,)),
            pltpu.SemaphoreType.DMA((2,)),
        ],
        compiler_params=_CompilerParams(vmem_limit_bytes=VLIM),
    )(x2, Wq, K_ext, V_ext)

    out2 = pl.pallas_call(
        _ar_body,
        out_shape=jax.ShapeDtypeStruct((SQ, DMODEL), jnp.float32),
        in_specs=[
            pl.BlockSpec(memory_space=pltpu.MemorySpace.VMEM),
            pl.BlockSpec(memory_space=pltpu.MemorySpace.VMEM),
        ],
        out_specs=pl.BlockSpec(memory_space=pltpu.MemorySpace.VMEM),
        scratch_shapes=[
            pltpu.VMEM((SQ, DMODEL), jnp.float32),
            pltpu.VMEM((2, CHUNK, HALF), jnp.bfloat16),
            pltpu.VMEM((2, CHUNK, HALF), jnp.bfloat16),
            pltpu.VMEM((NSTEP, CHUNK, HALF), jnp.bfloat16),
            pltpu.VMEM((NSTEP, CHUNK, HALF), jnp.bfloat16),
            pltpu.VMEM((NSTEP, CHUNK, HALF), jnp.bfloat16),
            pltpu.VMEM((NSTEP, CHUNK, HALF), jnp.bfloat16),
            pltpu.VMEM((CHUNK, HALF), jnp.bfloat16),
            pltpu.VMEM((CHUNK, HALF), jnp.bfloat16),
            pltpu.SemaphoreType.DMA((2,)),
            pltpu.SemaphoreType.DMA((2,)),
            pltpu.SemaphoreType.DMA((NSTEP,)),
            pltpu.SemaphoreType.DMA((NSTEP,)),
            pltpu.SemaphoreType.DMA((NSTEP,)),
            pltpu.SemaphoreType.DMA((NSTEP,)),
        ],
        compiler_params=_CompilerParams(
            collective_id=0, vmem_limit_bytes=VLIM
        ),
    )(ctx, Wo.reshape(H_PER, DH, DMODEL))

    return out2.reshape(1, SQ, DMODEL)


# device time: 181804 ns/iter; 1.0830x vs baseline; 1.0038x over previous
import functools

import jax
import jax.numpy as jnp
from jax import lax
from jax.experimental import pallas as pl
from jax.experimental.pallas import tpu as pltpu

_DeviceIdType = getattr(pl, "DeviceIdType", None) or pltpu.DeviceIdType
_sem_signal = getattr(pl, "semaphore_signal", None) or pltpu.semaphore_signal
_sem_wait = getattr(pl, "semaphore_wait", None) or pltpu.semaphore_wait
_run_scoped = getattr(pl, "run_scoped", None) or pltpu.run_scoped
_CompilerParams = getattr(pltpu, "CompilerParams", None) or getattr(
    pltpu, "TPUCompilerParams"
)

N_DEV = 16
H_PER = 8
SQ = 2048
SKV = 2048
DH = 128
DMODEL = 1024
SCALE = 0.08838834764831843
QT = 512
CHUNK = SQ // N_DEV
HALF = DMODEL // 2
NSTEP = N_DEV - 1
VLIM = 100 * 1024 * 1024


def _ring_pos(p):
    z = p // 4
    k = lax.rem(p, 4)
    return jnp.where(
        k == 0, z,
        jnp.where(k == 3, 7 - z, jnp.where(k == 2, 8 + z, 15 - z)),
    )


def _ring_to_log(rr):
    seg = rr // 4
    z = jnp.where(
        seg == 0, rr,
        jnp.where(seg == 1, 7 - rr, jnp.where(seg == 2, rr - 8, 15 - rr)),
    )
    k = jnp.where(
        seg == 0, 0, jnp.where(seg == 1, 3, jnp.where(seg == 2, 2, 1))
    )
    return 4 * z + k


def _attn_body(x_ref, wq_ref, k_hbm, v_hbm, ctx_ref, kbuf, vbuf, sk, sv):
    h = pl.program_id(0)
    p = lax.axis_index("i")
    ghead = p * H_PER + h
    slot = lax.rem(h, 2)
    nslot = lax.rem(h + 1, 2)

    def copies(head, sl):
        ck = pltpu.make_async_copy(k_hbm.at[0, :, head, :], kbuf.at[sl], sk.at[sl])
        cv = pltpu.make_async_copy(v_hbm.at[0, :, head, :], vbuf.at[sl], sv.at[sl])
        return ck, cv

    @pl.when(h == 0)
    def _():
        a, b = copies(ghead, slot)
        a.start()
        b.start()

    @pl.when(h < H_PER - 1)
    def _():
        a, b = copies(ghead + 1, nslot)
        a.start()
        b.start()

    q = jnp.dot(x_ref[...], wq_ref[...], preferred_element_type=jnp.float32)

    a, b = copies(ghead, slot)
    a.wait()
    b.wait()
    k = kbuf[slot].astype(jnp.bfloat16)
    v = vbuf[slot].astype(jnp.bfloat16)
    k0 = k[0:64]
    v0 = v[0:64]
    q = q.astype(jnp.bfloat16)

    def dotT(a_, b_):
        return lax.dot_general(a_, b_, (((1,), (1,)), ((), ())),
                               preferred_element_type=jnp.float32)

    for rho in range(3):
        qbs = [bq for bq in range(SQ // 64) if bq % 3 == rho]
        kbs = [bk for bk in range(SKV // 64) if bk % 3 == (3 - rho) % 3]
        qr = jnp.concatenate([q[bq * 64:(bq + 1) * 64] for bq in qbs], axis=0)
        ks = jnp.concatenate([k[bk * 64:(bk + 1) * 64] for bk in kbs], axis=0)
        vs = jnp.concatenate([v[bk * 64:(bk + 1) * 64] for bk in kbs], axis=0)
        parts = [dotT(qr, ks) * SCALE]
        if rho != 0:
            parts.append(dotT(qr, k0) * SCALE)
            parts.append(jnp.concatenate(
                [dotT(q[bq * 64:(bq + 1) * 64], k[bq * 64:(bq + 1) * 64])
                 for bq in qbs], axis=0) * SCALE)
        s = jnp.concatenate(parts, axis=1)
        m = jnp.max(s, axis=1, keepdims=True)
        w = jnp.exp(s - m)
        w = (w / jnp.sum(w, axis=1, keepdims=True)).astype(jnp.bfloat16)
        ncs = len(kbs) * 64
        ctx_main = jnp.dot(w[:, 0:ncs], vs, preferred_element_type=jnp.float32)
        if rho != 0:
            ctx_main = ctx_main + jnp.dot(w[:, ncs:ncs + 64], v0,
                                          preferred_element_type=jnp.float32)
            wd = w[:, ncs + 64:]
            for i, bq in enumerate(qbs):
                ctx_ref[0, bq * 64:(bq + 1) * 64, :] = (
                    ctx_main[i * 64:(i + 1) * 64]
                    + jnp.dot(wd[i * 64:(i + 1) * 64],
                              v[bq * 64:(bq + 1) * 64],
                              preferred_element_type=jnp.float32))
        else:
            for i, bq in enumerate(qbs):
                ctx_ref[0, bq * 64:(bq + 1) * 64, :] = \
                    ctx_main[i * 64:(i + 1) * 64]


def _ar_body(ctx_ref, wo_ref, out_ref, part,
             sbR, sbL, rsR, rsL, agR, agL, abR, abL,
             ssR, ssL, rsmR, rsmL, agmR, agmL):
    p = lax.axis_index("i")
    r = _ring_pos(p)
    nxt = _ring_to_log(lax.rem(r + 1, N_DEV))
    prv = _ring_to_log(lax.rem(r + N_DEV - 1, N_DEV))

    barrier = pltpu.get_barrier_semaphore()
    for nbr in (prv, nxt):
        _sem_signal(barrier, inc=1, device_id=(nbr,),
                    device_id_type=_DeviceIdType.MESH)
    _sem_wait(barrier, 2)

    def rows(c):
        return pl.ds(c * CHUNK, CHUNK)

    def proj_chunk(c):
        rws = rows(c)
        accc = jnp.dot(ctx_ref[0, rws, :], wo_ref[0],
                       preferred_element_type=jnp.float32)
        for h in range(1, H_PER):
            accc = accc + jnp.dot(ctx_ref[h, rws, :], wo_ref[h],
                                  preferred_element_type=jnp.float32)
        part[rws, :] = accc

    proj_chunk(r)

    def cR(s):
        return lax.rem(r - s + N_DEV, N_DEV)

    def cL(s):
        return lax.rem(r + s, N_DEV)

    def rs_rdma(s, right):
        if right:
            return pltpu.make_async_remote_copy(
                src_ref=sbR.at[s % 2], dst_ref=rsR.at[s],
                send_sem=ssR.at[s % 2], recv_sem=rsmR.at[s],
                device_id=(nxt,), device_id_type=_DeviceIdType.MESH)
        return pltpu.make_async_remote_copy(
            src_ref=sbL.at[s % 2], dst_ref=rsL.at[s],
            send_sem=ssL.at[s % 2], recv_sem=rsmL.at[s],
            device_id=(prv,), device_id_type=_DeviceIdType.MESH)

    sbR[0] = part[rows(cR(0)), 0:HALF].astype(jnp.bfloat16)
    sbL[0] = part[rows(cL(0)), HALF:DMODEL].astype(jnp.bfloat16)
    rs_rdma(0, True).start()
    rs_rdma(0, False).start()
    for s in range(1, NSTEP):
        if s <= 7:
            proj_chunk(lax.rem(r - s + N_DEV, N_DEV))
            proj_chunk(lax.rem(r + s, N_DEV))
        elif s == 8:
            proj_chunk(lax.rem(r + 8, N_DEV))
        rs_rdma(s - 1, True).wait_recv()
        rs_rdma(s - 1, False).wait_recv()
        if s >= 2:
            rs_rdma(s - 2, True).wait_send()
            rs_rdma(s - 2, False).wait_send()
        sbR[s % 2] = (rsR[s - 1].astype(jnp.float32)
                      + part[rows(cR(s)), 0:HALF]).astype(jnp.bfloat16)
        sbL[s % 2] = (rsL[s - 1].astype(jnp.float32)
                      + part[rows(cL(s)), HALF:DMODEL]).astype(jnp.bfloat16)
        rs_rdma(s, True).start()
        rs_rdma(s, False).start()
    rs_rdma(NSTEP - 1, True).wait_recv()
    rs_rdma(NSTEP - 1, False).wait_recv()
    for s in (NSTEP - 2, NSTEP - 1):
        rs_rdma(s, True).wait_send()
        rs_rdma(s, False).wait_send()

    redR = rsR[NSTEP - 1].astype(jnp.float32) + part[rows(cR(NSTEP)), 0:HALF]
    redL = (rsL[NSTEP - 1].astype(jnp.float32)
            + part[rows(cL(NSTEP)), HALF:DMODEL])
    abR[...] = redR.astype(jnp.bfloat16)
    abL[...] = redL.astype(jnp.bfloat16)
    out_ref[rows(cR(NSTEP)), 0:HALF] = redR
    out_ref[rows(cL(NSTEP)), HALF:DMODEL] = redL

    def ag_rdma(s, right):
        if right:
            return pltpu.make_async_remote_copy(
                src_ref=(abR if s == 0 else agR.at[s - 1]), dst_ref=agR.at[s],
                send_sem=ssR.at[s % 2], recv_sem=agmR.at[s],
                device_id=(nxt,), device_id_type=_DeviceIdType.MESH)
        return pltpu.make_async_remote_copy(
            src_ref=(abL if s == 0 else agL.at[s - 1]), dst_ref=agL.at[s],
            send_sem=ssL.at[s % 2], recv_sem=agmL.at[s],
            device_id=(prv,), device_id_type=_DeviceIdType.MESH)

    ag_rdma(0, True).start()
    ag_rdma(0, False).start()
    for s in range(NSTEP):
        ag_rdma(s, True).wait_recv()
        ag_rdma(s, False).wait_recv()
        if s < NSTEP - 1:
            if s >= 1:
                ag_rdma(s - 1, True).wait_send()
                ag_rdma(s - 1, False).wait_send()
            ag_rdma(s + 1, True).start()
            ag_rdma(s + 1, False).start()
        out_ref[rows(cR(s)), 0:HALF] = agR[s].astype(jnp.float32)
        out_ref[rows(cL(s)), HALF:DMODEL] = agL[s].astype(jnp.float32)
    for s in (NSTEP - 2, NSTEP - 1):
        ag_rdma(s, True).wait_send()
        ag_rdma(s, False).wait_send()

    @functools.partial(_run_scoped, sem2=pltpu.SemaphoreType.REGULAR)
    def _(sem2):
        for nbr in (prv, nxt):
            _sem_signal(sem2, inc=1, device_id=(nbr,),
                        device_id_type=_DeviceIdType.MESH)
        _sem_wait(sem2, 2)


def kernel(x, Wq, K_ext, V_ext, Wo):
    x2 = x.reshape(SQ, DMODEL)

    ctx = pl.pallas_call(
        _attn_body,
        grid=(H_PER,),
        out_shape=jax.ShapeDtypeStruct((H_PER, SQ, DH), jnp.float32),
        in_specs=[
            pl.BlockSpec((SQ, DMODEL), lambda h: (0, 0)),
            pl.BlockSpec((DMODEL, DH), lambda h: (0, h)),
            pl.BlockSpec(memory_space=pl.ANY),
            pl.BlockSpec(memory_space=pl.ANY),
        ],
        out_specs=pl.BlockSpec((1, SQ, DH), lambda h: (h, 0, 0)),
        scratch_shapes=[
            pltpu.VMEM((2, SKV, DH), jnp.float32),
            pltpu.VMEM((2, SKV, DH), jnp.float32),
            pltpu.SemaphoreType.DMA((2,)),
            pltpu.SemaphoreType.DMA((2,)),
        ],
        compiler_params=_CompilerParams(vmem_limit_bytes=VLIM),
    )(x2, Wq, K_ext, V_ext)

    out2 = pl.pallas_call(
        _ar_body,
        out_shape=jax.ShapeDtypeStruct((SQ, DMODEL), jnp.float32),
        in_specs=[
            pl.BlockSpec(memory_space=pltpu.MemorySpace.VMEM),
            pl.BlockSpec(memory_space=pltpu.MemorySpace.VMEM),
        ],
        out_specs=pl.BlockSpec(memory_space=pltpu.MemorySpace.VMEM),
        scratch_shapes=[
            pltpu.VMEM((SQ, DMODEL), jnp.float32),
            pltpu.VMEM((2, CHUNK, HALF), jnp.bfloat16),
            pltpu.VMEM((2, CHUNK, HALF), jnp.bfloat16),
            pltpu.VMEM((NSTEP, CHUNK, HALF), jnp.bfloat16),
            pltpu.VMEM((NSTEP, CHUNK, HALF), jnp.bfloat16),
            pltpu.VMEM((NSTEP, CHUNK, HALF), jnp.bfloat16),
            pltpu.VMEM((NSTEP, CHUNK, HALF), jnp.bfloat16),
            pltpu.VMEM((CHUNK, HALF), jnp.bfloat16),
            pltpu.VMEM((CHUNK, HALF), jnp.bfloat16),
            pltpu.SemaphoreType.DMA((2,)),
            pltpu.SemaphoreType.DMA((2,)),
            pltpu.SemaphoreType.DMA((NSTEP,)),
            pltpu.SemaphoreType.DMA((NSTEP,)),
            pltpu.SemaphoreType.DMA((NSTEP,)),
            pltpu.SemaphoreType.DMA((NSTEP,)),
        ],
        compiler_params=_CompilerParams(
            collective_id=0, vmem_limit_bytes=VLIM
        ),
    )(ctx, Wo.reshape(H_PER, DH, DMODEL))

    return out2.reshape(1, SQ, DMODEL)


# device time: 174678 ns/iter; 1.1272x vs baseline; 1.0408x over previous
import functools

import jax
import jax.numpy as jnp
from jax import lax
from jax.experimental import pallas as pl
from jax.experimental.pallas import tpu as pltpu

_DeviceIdType = getattr(pl, "DeviceIdType", None) or pltpu.DeviceIdType
_sem_signal = getattr(pl, "semaphore_signal", None) or pltpu.semaphore_signal
_sem_wait = getattr(pl, "semaphore_wait", None) or pltpu.semaphore_wait
_run_scoped = getattr(pl, "run_scoped", None) or pltpu.run_scoped
_CompilerParams = getattr(pltpu, "CompilerParams", None) or getattr(
    pltpu, "TPUCompilerParams"
)

N_DEV = 16
H_PER = 8
SQ = 2048
SKV = 2048
DH = 128
DMODEL = 1024
SCALE = 0.08838834764831843
QT = 512
CHUNK = SQ // N_DEV
HALF = DMODEL // 2
NSTEP = N_DEV - 1
VLIM = 100 * 1024 * 1024


def _ring_pos(p):
    z = p // 4
    k = lax.rem(p, 4)
    return jnp.where(
        k == 0, z,
        jnp.where(k == 3, 7 - z, jnp.where(k == 2, 8 + z, 15 - z)),
    )


def _ring_to_log(rr):
    seg = rr // 4
    z = jnp.where(
        seg == 0, rr,
        jnp.where(seg == 1, 7 - rr, jnp.where(seg == 2, rr - 8, 15 - rr)),
    )
    k = jnp.where(
        seg == 0, 0, jnp.where(seg == 1, 3, jnp.where(seg == 2, 2, 1))
    )
    return 4 * z + k


_R0 = [b for b in range(32) if b % 3 == 0]
_R1 = [b for b in range(32) if b % 3 == 1]
_R2 = [b for b in range(32) if b % 3 == 2]
_KORD = _R0 + _R2 + _R1
_QORD = _R0 + _R1 + _R2
_KOFF = {0: 0, 1: 704, 2: 1344}
_KLEN = {0: 704, 1: 640, 2: 704}
_QOFF = {0: 0, 1: 704, 2: 1408}
_QBS = {0: _R0, 1: _R1, 2: _R2}
_DOFF = {1: 1344, 2: 704}


def _attn_body(x_ref, wq_ref, k_hbm, v_hbm, ctx_ref, xp, kbuf, vbuf, sk, sv):
    h = pl.program_id(0)
    p = lax.axis_index("i")
    ghead = p * H_PER + h
    slot = lax.rem(h, 2)
    nslot = lax.rem(h + 1, 2)

    def copy_ops(head, sl):
        ops = []
        for i, b in enumerate(_KORD):
            r = slice(i * 64, (i + 1) * 64)
            ops.append(pltpu.make_async_copy(
                k_hbm.at[0, b * 64:(b + 1) * 64, head, :],
                kbuf.at[sl, r, :], sk.at[sl, i]))
            ops.append(pltpu.make_async_copy(
                v_hbm.at[0, b * 64:(b + 1) * 64, head, :],
                vbuf.at[sl, r, :], sv.at[sl, i]))
        return ops

    @pl.when(h == 0)
    def _():
        for op in copy_ops(ghead, slot):
            op.start()

    @pl.when(h == 0)
    def _():
        for i, b in enumerate(_QORD):
            xp[i * 64:(i + 1) * 64, :] = x_ref[b * 64:(b + 1) * 64, :]

    @pl.when(h < H_PER - 1)
    def _():
        for op in copy_ops(ghead + 1, nslot):
            op.start()

    q = jnp.dot(xp[...], wq_ref[...], preferred_element_type=jnp.float32)

    for op in copy_ops(ghead, slot):
        op.wait()
    k = kbuf[slot].astype(jnp.bfloat16)
    v = vbuf[slot].astype(jnp.bfloat16)
    k0 = k[0:64]
    v0 = v[0:64]
    q = q.astype(jnp.bfloat16)

    def dotT(a_, b_):
        return lax.dot_general(a_, b_, (((1,), (1,)), ((), ())),
                               preferred_element_type=jnp.float32)

    for rho in range(3):
        qbs = _QBS[rho]
        qo = _QOFF[rho]
        nq = len(qbs) * 64
        qr = q[qo:qo + nq]
        ks = k[_KOFF[rho]:_KOFF[rho] + _KLEN[rho]]
        vs = v[_KOFF[rho]:_KOFF[rho] + _KLEN[rho]]
        parts = [dotT(qr, ks) * SCALE]
        if rho != 0:
            do = _DOFF[rho]
            parts.append(dotT(qr, k0) * SCALE)
            parts.append(jnp.concatenate(
                [dotT(qr[i * 64:(i + 1) * 64],
                      k[do + i * 64:do + (i + 1) * 64])
                 for i in range(len(qbs))], axis=0) * SCALE)
        s = jnp.concatenate(parts, axis=1)
        e = jnp.exp(s)
        denom = jnp.sum(e, axis=1, keepdims=True)
        w = e.astype(jnp.bfloat16)
        ncs = _KLEN[rho]
        ctx_main = jnp.dot(w[:, 0:ncs], vs, preferred_element_type=jnp.float32)
        if rho != 0:
            do = _DOFF[rho]
            ctx_main = ctx_main + jnp.dot(w[:, ncs:ncs + 64], v0,
                                          preferred_element_type=jnp.float32)
            wd = w[:, ncs + 64:]
            for i, bq in enumerate(qbs):
                ctx_ref[0, bq * 64:(bq + 1) * 64, :] = (
                    (ctx_main[i * 64:(i + 1) * 64]
                     + jnp.dot(wd[i * 64:(i + 1) * 64],
                               v[do + i * 64:do + (i + 1) * 64],
                               preferred_element_type=jnp.float32))
                    / denom[i * 64:(i + 1) * 64])
        else:
            for i, bq in enumerate(qbs):
                ctx_ref[0, bq * 64:(bq + 1) * 64, :] = (
                    ctx_main[i * 64:(i + 1) * 64]
                    / denom[i * 64:(i + 1) * 64])


def _ar_body(ctx_ref, wo_ref, out_ref, part,
             sbR, sbL, rsR, rsL, agR, agL, abR, abL,
             ssR, ssL, rsmR, rsmL, agmR, agmL):
    p = lax.axis_index("i")
    r = _ring_pos(p)
    nxt = _ring_to_log(lax.rem(r + 1, N_DEV))
    prv = _ring_to_log(lax.rem(r + N_DEV - 1, N_DEV))

    barrier = pltpu.get_barrier_semaphore()
    for nbr in (prv, nxt):
        _sem_signal(barrier, inc=1, device_id=(nbr,),
                    device_id_type=_DeviceIdType.MESH)
    _sem_wait(barrier, 2)

    def rows(c):
        return pl.ds(c * CHUNK, CHUNK)

    def proj_chunk(c):
        rws = rows(c)
        accc = jnp.dot(ctx_ref[0, rws, :], wo_ref[0],
                       preferred_element_type=jnp.float32)
        for h in range(1, H_PER):
            accc = accc + jnp.dot(ctx_ref[h, rws, :], wo_ref[h],
                                  preferred_element_type=jnp.float32)
        part[rws, :] = accc

    proj_chunk(r)

    def cR(s):
        return lax.rem(r - s + N_DEV, N_DEV)

    def cL(s):
        return lax.rem(r + s, N_DEV)

    def rs_rdma(s, right):
        if right:
            return pltpu.make_async_remote_copy(
                src_ref=sbR.at[s % 2], dst_ref=rsR.at[s],
                send_sem=ssR.at[s % 2], recv_sem=rsmR.at[s],
                device_id=(nxt,), device_id_type=_DeviceIdType.MESH)
        return pltpu.make_async_remote_copy(
            src_ref=sbL.at[s % 2], dst_ref=rsL.at[s],
            send_sem=ssL.at[s % 2], recv_sem=rsmL.at[s],
            device_id=(prv,), device_id_type=_DeviceIdType.MESH)

    sbR[0] = part[rows(cR(0)), 0:HALF].astype(jnp.bfloat16)
    sbL[0] = part[rows(cL(0)), HALF:DMODEL].astype(jnp.bfloat16)
    rs_rdma(0, True).start()
    rs_rdma(0, False).start()
    for s in range(1, NSTEP):
        if s <= 7:
            proj_chunk(lax.rem(r - s + N_DEV, N_DEV))
            proj_chunk(lax.rem(r + s, N_DEV))
        elif s == 8:
            proj_chunk(lax.rem(r + 8, N_DEV))
        rs_rdma(s - 1, True).wait_recv()
        rs_rdma(s - 1, False).wait_recv()
        if s >= 2:
            rs_rdma(s - 2, True).wait_send()
            rs_rdma(s - 2, False).wait_send()
        sbR[s % 2] = (rsR[s - 1].astype(jnp.float32)
                      + part[rows(cR(s)), 0:HALF]).astype(jnp.bfloat16)
        sbL[s % 2] = (rsL[s - 1].astype(jnp.float32)
                      + part[rows(cL(s)), HALF:DMODEL]).astype(jnp.bfloat16)
        rs_rdma(s, True).start()
        rs_rdma(s, False).start()
    rs_rdma(NSTEP - 1, True).wait_recv()
    rs_rdma(NSTEP - 1, False).wait_recv()
    for s in (NSTEP - 2, NSTEP - 1):
        rs_rdma(s, True).wait_send()
        rs_rdma(s, False).wait_send()

    redR = rsR[NSTEP - 1].astype(jnp.float32) + part[rows(cR(NSTEP)), 0:HALF]
    redL = (rsL[NSTEP - 1].astype(jnp.float32)
            + part[rows(cL(NSTEP)), HALF:DMODEL])
    abR[...] = redR.astype(jnp.bfloat16)
    abL[...] = redL.astype(jnp.bfloat16)
    out_ref[rows(cR(NSTEP)), 0:HALF] = redR
    out_ref[rows(cL(NSTEP)), HALF:DMODEL] = redL

    def ag_rdma(s, right):
        if right:
            return pltpu.make_async_remote_copy(
                src_ref=(abR if s == 0 else agR.at[s - 1]), dst_ref=agR.at[s],
                send_sem=ssR.at[s % 2], recv_sem=agmR.at[s],
                device_id=(nxt,), device_id_type=_DeviceIdType.MESH)
        return pltpu.make_async_remote_copy(
            src_ref=(abL if s == 0 else agL.at[s - 1]), dst_ref=agL.at[s],
            send_sem=ssL.at[s % 2], recv_sem=agmL.at[s],
            device_id=(prv,), device_id_type=_DeviceIdType.MESH)

    ag_rdma(0, True).start()
    ag_rdma(0, False).start()
    for s in range(NSTEP):
        ag_rdma(s, True).wait_recv()
        ag_rdma(s, False).wait_recv()
        if s < NSTEP - 1:
            if s >= 1:
                ag_rdma(s - 1, True).wait_send()
                ag_rdma(s - 1, False).wait_send()
            ag_rdma(s + 1, True).start()
            ag_rdma(s + 1, False).start()
        out_ref[rows(cR(s)), 0:HALF] = agR[s].astype(jnp.float32)
        out_ref[rows(cL(s)), HALF:DMODEL] = agL[s].astype(jnp.float32)
    for s in (NSTEP - 2, NSTEP - 1):
        ag_rdma(s, True).wait_send()
        ag_rdma(s, False).wait_send()

    @functools.partial(_run_scoped, sem2=pltpu.SemaphoreType.REGULAR)
    def _(sem2):
        for nbr in (prv, nxt):
            _sem_signal(sem2, inc=1, device_id=(nbr,),
                        device_id_type=_DeviceIdType.MESH)
        _sem_wait(sem2, 2)


def kernel(x, Wq, K_ext, V_ext, Wo):
    x2 = x.reshape(SQ, DMODEL)

    ctx = pl.pallas_call(
        _attn_body,
        grid=(H_PER,),
        out_shape=jax.ShapeDtypeStruct((H_PER, SQ, DH), jnp.float32),
        in_specs=[
            pl.BlockSpec((SQ, DMODEL), lambda h: (0, 0)),
            pl.BlockSpec((DMODEL, DH), lambda h: (0, h)),
            pl.BlockSpec(memory_space=pl.ANY),
            pl.BlockSpec(memory_space=pl.ANY),
        ],
        out_specs=pl.BlockSpec((1, SQ, DH), lambda h: (h, 0, 0)),
        scratch_shapes=[
            pltpu.VMEM((SQ, DMODEL), jnp.float32),
            pltpu.VMEM((2, SKV, DH), jnp.float32),
            pltpu.VMEM((2, SKV, DH), jnp.float32),
            pltpu.SemaphoreType.DMA((2, 32)),
            pltpu.SemaphoreType.DMA((2, 32)),
        ],
        compiler_params=_CompilerParams(vmem_limit_bytes=VLIM),
    )(x2, Wq, K_ext, V_ext)

    out2 = pl.pallas_call(
        _ar_body,
        out_shape=jax.ShapeDtypeStruct((SQ, DMODEL), jnp.float32),
        in_specs=[
            pl.BlockSpec(memory_space=pltpu.MemorySpace.VMEM),
            pl.BlockSpec(memory_space=pltpu.MemorySpace.VMEM),
        ],
        out_specs=pl.BlockSpec(memory_space=pltpu.MemorySpace.VMEM),
        scratch_shapes=[
            pltpu.VMEM((SQ, DMODEL), jnp.float32),
            pltpu.VMEM((2, CHUNK, HALF), jnp.bfloat16),
            pltpu.VMEM((2, CHUNK, HALF), jnp.bfloat16),
            pltpu.VMEM((NSTEP, CHUNK, HALF), jnp.bfloat16),
            pltpu.VMEM((NSTEP, CHUNK, HALF), jnp.bfloat16),
            pltpu.VMEM((NSTEP, CHUNK, HALF), jnp.bfloat16),
            pltpu.VMEM((NSTEP, CHUNK, HALF), jnp.bfloat16),
            pltpu.VMEM((CHUNK, HALF), jnp.bfloat16),
            pltpu.VMEM((CHUNK, HALF), jnp.bfloat16),
            pltpu.SemaphoreType.DMA((2,)),
            pltpu.SemaphoreType.DMA((2,)),
            pltpu.SemaphoreType.DMA((NSTEP,)),
            pltpu.SemaphoreType.DMA((NSTEP,)),
            pltpu.SemaphoreType.DMA((NSTEP,)),
            pltpu.SemaphoreType.DMA((NSTEP,)),
        ],
        compiler_params=_CompilerParams(
            collective_id=0, vmem_limit_bytes=VLIM
        ),
    )(ctx, Wo.reshape(H_PER, DH, DMODEL))

    return out2.reshape(1, SQ, DMODEL)


# device time: 174677 ns/iter; 1.1272x vs baseline; 1.0000x over previous
import functools

import jax
import jax.numpy as jnp
from jax import lax
from jax.experimental import pallas as pl
from jax.experimental.pallas import tpu as pltpu

_DeviceIdType = getattr(pl, "DeviceIdType", None) or pltpu.DeviceIdType
_sem_signal = getattr(pl, "semaphore_signal", None) or pltpu.semaphore_signal
_sem_wait = getattr(pl, "semaphore_wait", None) or pltpu.semaphore_wait
_run_scoped = getattr(pl, "run_scoped", None) or pltpu.run_scoped
_CompilerParams = getattr(pltpu, "CompilerParams", None) or getattr(
    pltpu, "TPUCompilerParams"
)

N_DEV = 16
H_PER = 8
SQ = 2048
SKV = 2048
DH = 128
DMODEL = 1024
SCALE = 0.08838834764831843
QT = 512
CHUNK = SQ // N_DEV
HALF = DMODEL // 2
NSTEP = N_DEV - 1
VLIM = 100 * 1024 * 1024


def _ring_pos(p):
    z = p // 4
    k = lax.rem(p, 4)
    return jnp.where(
        k == 0, z,
        jnp.where(k == 3, 7 - z, jnp.where(k == 2, 8 + z, 15 - z)),
    )


def _ring_to_log(rr):
    seg = rr // 4
    z = jnp.where(
        seg == 0, rr,
        jnp.where(seg == 1, 7 - rr, jnp.where(seg == 2, rr - 8, 15 - rr)),
    )
    k = jnp.where(
        seg == 0, 0, jnp.where(seg == 1, 3, jnp.where(seg == 2, 2, 1))
    )
    return 4 * z + k


_R0 = [b for b in range(32) if b % 3 == 0]
_R1 = [b for b in range(32) if b % 3 == 1]
_R2 = [b for b in range(32) if b % 3 == 2]
_KORD = _R0 + _R2 + _R1
_QORD = _R0 + _R1 + _R2
_KOFF = {0: 0, 1: 704, 2: 1344}
_KLEN = {0: 704, 1: 640, 2: 704}
_QOFF = {0: 0, 1: 704, 2: 1408}
_QBS = {0: _R0, 1: _R1, 2: _R2}
_DOFF = {1: 1344, 2: 704}


def _attn_body(x_ref, wq_ref, k_hbm, v_hbm, ctx_ref, xp, kbuf, vbuf, sk, sv):
    h = pl.program_id(0)
    p = lax.axis_index("i")
    ghead = p * H_PER + h
    slot = lax.rem(h, 2)
    nslot = lax.rem(h + 1, 2)

    def copy_ops(head, sl):
        ops = []
        for i, b in enumerate(_KORD):
            r = slice(i * 64, (i + 1) * 64)
            ops.append(pltpu.make_async_copy(
                k_hbm.at[0, b * 64:(b + 1) * 64, head, :],
                kbuf.at[sl, r, :], sk.at[sl, i]))
            ops.append(pltpu.make_async_copy(
                v_hbm.at[0, b * 64:(b + 1) * 64, head, :],
                vbuf.at[sl, r, :], sv.at[sl, i]))
        return ops

    @pl.when(h == 0)
    def _():
        for op in copy_ops(ghead, slot):
            op.start()

    @pl.when(h == 0)
    def _():
        for i, b in enumerate(_QORD):
            xp[i * 64:(i + 1) * 64, :] = x_ref[b * 64:(b + 1) * 64, :]

    @pl.when(h < H_PER - 1)
    def _():
        for op in copy_ops(ghead + 1, nslot):
            op.start()

    q = jnp.dot(xp[...], wq_ref[...], preferred_element_type=jnp.float32)

    for op in copy_ops(ghead, slot):
        op.wait()
    k = kbuf[slot].astype(jnp.bfloat16)
    v = vbuf[slot].astype(jnp.bfloat16)
    k0 = k[0:64]
    v0 = v[0:64]
    q = q.astype(jnp.bfloat16)

    def dotT(a_, b_):
        return lax.dot_general(a_, b_, (((1,), (1,)), ((), ())),
                               preferred_element_type=jnp.float32)

    for rho in range(3):
        qbs = _QBS[rho]
        qo = _QOFF[rho]
        nq = len(qbs) * 64
        qr = q[qo:qo + nq]
        ks = k[_KOFF[rho]:_KOFF[rho] + _KLEN[rho]]
        vs = v[_KOFF[rho]:_KOFF[rho] + _KLEN[rho]]
        parts = [dotT(qr, ks) * SCALE]
        if rho != 0:
            do = _DOFF[rho]
            parts.append(dotT(qr, k0) * SCALE)
            parts.append(jnp.concatenate(
                [dotT(qr[i * 64:(i + 1) * 64],
                      k[do + i * 64:do + (i + 1) * 64])
                 for i in range(len(qbs))], axis=0) * SCALE)
        s = jnp.concatenate(parts, axis=1)
        e = jnp.exp(s)
        denom = jnp.sum(e, axis=1, keepdims=True)
        w = e.astype(jnp.bfloat16)
        ncs = _KLEN[rho]
        ctx_main = jnp.dot(w[:, 0:ncs], vs, preferred_element_type=jnp.float32)
        if rho != 0:
            do = _DOFF[rho]
            ctx_main = ctx_main + jnp.dot(w[:, ncs:ncs + 64], v0,
                                          preferred_element_type=jnp.float32)
            wd = w[:, ncs + 64:]
            for i, bq in enumerate(qbs):
                ctx_ref[0, bq * 64:(bq + 1) * 64, :] = (
                    (ctx_main[i * 64:(i + 1) * 64]
                     + jnp.dot(wd[i * 64:(i + 1) * 64],
                               v[do + i * 64:do + (i + 1) * 64],
                               preferred_element_type=jnp.float32))
                    / denom[i * 64:(i + 1) * 64])
        else:
            for i, bq in enumerate(qbs):
                ctx_ref[0, bq * 64:(bq + 1) * 64, :] = (
                    ctx_main[i * 64:(i + 1) * 64]
                    / denom[i * 64:(i + 1) * 64])


def _ar_body(ctx_ref, wo_ref, out_ref, part,
             sbR, sbL, rsR, rsL, agR, agL, abR, abL,
             ssR, ssL, rsmR, rsmL, agmR, agmL):
    p = lax.axis_index("i")
    r = _ring_pos(p)
    nxt = _ring_to_log(lax.rem(r + 1, N_DEV))
    prv = _ring_to_log(lax.rem(r + N_DEV - 1, N_DEV))

    barrier = pltpu.get_barrier_semaphore()
    for nbr in (prv, nxt):
        _sem_signal(barrier, inc=1, device_id=(nbr,),
                    device_id_type=_DeviceIdType.MESH)
    _sem_wait(barrier, 2)

    def rows(c):
        return pl.ds(c * CHUNK, CHUNK)

    def proj_half(c, left):
        rws = rows(c)
        cols = slice(0, HALF) if left else slice(HALF, DMODEL)
        accc = jnp.dot(ctx_ref[0, rws, :], wo_ref[0, :, cols],
                       preferred_element_type=jnp.float32)
        for h in range(1, H_PER):
            accc = accc + jnp.dot(ctx_ref[h, rws, :], wo_ref[h, :, cols],
                                  preferred_element_type=jnp.float32)
        part[rws, cols] = accc

    proj_half(r, True)
    proj_half(r, False)

    def cR(s):
        return lax.rem(r - s + N_DEV, N_DEV)

    def cL(s):
        return lax.rem(r + s, N_DEV)

    def rs_rdma(s, right):
        if right:
            return pltpu.make_async_remote_copy(
                src_ref=sbR.at[s % 2], dst_ref=rsR.at[s],
                send_sem=ssR.at[s % 2], recv_sem=rsmR.at[s],
                device_id=(nxt,), device_id_type=_DeviceIdType.MESH)
        return pltpu.make_async_remote_copy(
            src_ref=sbL.at[s % 2], dst_ref=rsL.at[s],
            send_sem=ssL.at[s % 2], recv_sem=rsmL.at[s],
            device_id=(prv,), device_id_type=_DeviceIdType.MESH)

    sbR[0] = part[rows(cR(0)), 0:HALF].astype(jnp.bfloat16)
    sbL[0] = part[rows(cL(0)), HALF:DMODEL].astype(jnp.bfloat16)
    rs_rdma(0, True).start()
    rs_rdma(0, False).start()
    for s in range(1, NSTEP):
        proj_half(cR(s), True)
        proj_half(cL(s), False)
        rs_rdma(s - 1, True).wait_recv()
        rs_rdma(s - 1, False).wait_recv()
        if s >= 2:
            rs_rdma(s - 2, True).wait_send()
            rs_rdma(s - 2, False).wait_send()
        sbR[s % 2] = (rsR[s - 1].astype(jnp.float32)
                      + part[rows(cR(s)), 0:HALF]).astype(jnp.bfloat16)
        sbL[s % 2] = (rsL[s - 1].astype(jnp.float32)
                      + part[rows(cL(s)), HALF:DMODEL]).astype(jnp.bfloat16)
        rs_rdma(s, True).start()
        rs_rdma(s, False).start()
    proj_half(cR(NSTEP), True)
    proj_half(cL(NSTEP), False)
    rs_rdma(NSTEP - 1, True).wait_recv()
    rs_rdma(NSTEP - 1, False).wait_recv()
    for s in (NSTEP - 2, NSTEP - 1):
        rs_rdma(s, True).wait_send()
        rs_rdma(s, False).wait_send()

    redR = rsR[NSTEP - 1].astype(jnp.float32) + part[rows(cR(NSTEP)), 0:HALF]
    redL = (rsL[NSTEP - 1].astype(jnp.float32)
            + part[rows(cL(NSTEP)), HALF:DMODEL])
    abR[...] = redR.astype(jnp.bfloat16)
    abL[...] = redL.astype(jnp.bfloat16)
    out_ref[rows(cR(NSTEP)), 0:HALF] = redR
    out_ref[rows(cL(NSTEP)), HALF:DMODEL] = redL

    def ag_rdma(s, right):
        if right:
            return pltpu.make_async_remote_copy(
                src_ref=(abR if s == 0 else agR.at[s - 1]), dst_ref=agR.at[s],
                send_sem=ssR.at[s % 2], recv_sem=agmR.at[s],
                device_id=(nxt,), device_id_type=_DeviceIdType.MESH)
        return pltpu.make_async_remote_copy(
            src_ref=(abL if s == 0 else agL.at[s - 1]), dst_ref=agL.at[s],
            send_sem=ssL.at[s % 2], recv_sem=agmL.at[s],
            device_id=(prv,), device_id_type=_DeviceIdType.MESH)

    ag_rdma(0, True).start()
    ag_rdma(0, False).start()
    for s in range(NSTEP):
        ag_rdma(s, True).wait_recv()
        ag_rdma(s, False).wait_recv()
        if s < NSTEP - 1:
            if s >= 1:
                ag_rdma(s - 1, True).wait_send()
                ag_rdma(s - 1, False).wait_send()
            ag_rdma(s + 1, True).start()
            ag_rdma(s + 1, False).start()
        out_ref[rows(cR(s)), 0:HALF] = agR[s].astype(jnp.float32)
        out_ref[rows(cL(s)), HALF:DMODEL] = agL[s].astype(jnp.float32)
    for s in (NSTEP - 2, NSTEP - 1):
        ag_rdma(s, True).wait_send()
        ag_rdma(s, False).wait_send()

    @functools.partial(_run_scoped, sem2=pltpu.SemaphoreType.REGULAR)
    def _(sem2):
        for nbr in (prv, nxt):
            _sem_signal(sem2, inc=1, device_id=(nbr,),
                        device_id_type=_DeviceIdType.MESH)
        _sem_wait(sem2, 2)


def kernel(x, Wq, K_ext, V_ext, Wo):
    x2 = x.reshape(SQ, DMODEL)

    ctx = pl.pallas_call(
        _attn_body,
        grid=(H_PER,),
        out_shape=jax.ShapeDtypeStruct((H_PER, SQ, DH), jnp.float32),
        in_specs=[
            pl.BlockSpec((SQ, DMODEL), lambda h: (0, 0)),
            pl.BlockSpec((DMODEL, DH), lambda h: (0, h)),
            pl.BlockSpec(memory_space=pl.ANY),
            pl.BlockSpec(memory_space=pl.ANY),
        ],
        out_specs=pl.BlockSpec((1, SQ, DH), lambda h: (h, 0, 0)),
        scratch_shapes=[
            pltpu.VMEM((SQ, DMODEL), jnp.float32),
            pltpu.VMEM((2, SKV, DH), jnp.float32),
            pltpu.VMEM((2, SKV, DH), jnp.float32),
            pltpu.SemaphoreType.DMA((2, 32)),
            pltpu.SemaphoreType.DMA((2, 32)),
        ],
        compiler_params=_CompilerParams(vmem_limit_bytes=VLIM),
    )(x2, Wq, K_ext, V_ext)

    out2 = pl.pallas_call(
        _ar_body,
        out_shape=jax.ShapeDtypeStruct((SQ, DMODEL), jnp.float32),
        in_specs=[
            pl.BlockSpec(memory_space=pltpu.MemorySpace.VMEM),
            pl.BlockSpec(memory_space=pltpu.MemorySpace.VMEM),
        ],
        out_specs=pl.BlockSpec(memory_space=pltpu.MemorySpace.VMEM),
        scratch_shapes=[
            pltpu.VMEM((SQ, DMODEL), jnp.float32),
            pltpu.VMEM((2, CHUNK, HALF), jnp.bfloat16),
            pltpu.VMEM((2, CHUNK, HALF), jnp.bfloat16),
            pltpu.VMEM((NSTEP, CHUNK, HALF), jnp.bfloat16),
            pltpu.VMEM((NSTEP, CHUNK, HALF), jnp.bfloat16),
            pltpu.VMEM((NSTEP, CHUNK, HALF), jnp.bfloat16),
            pltpu.VMEM((NSTEP, CHUNK, HALF), jnp.bfloat16),
            pltpu.VMEM((CHUNK, HALF), jnp.bfloat16),
            pltpu.VMEM((CHUNK, HALF), jnp.bfloat16),
            pltpu.SemaphoreType.DMA((2,)),
            pltpu.SemaphoreType.DMA((2,)),
            pltpu.SemaphoreType.DMA((NSTEP,)),
            pltpu.SemaphoreType.DMA((NSTEP,)),
            pltpu.SemaphoreType.DMA((NSTEP,)),
            pltpu.SemaphoreType.DMA((NSTEP,)),
        ],
        compiler_params=_CompilerParams(
            collective_id=0, vmem_limit_bytes=VLIM
        ),
    )(ctx, Wo.reshape(H_PER, DH, DMODEL))

    return out2.reshape(1, SQ, DMODEL)
